# baseline (speedup 1.0000x reference)
import jax
import jax.numpy as jnp
from jax.experimental import pallas as pl
from jax.experimental.pallas import tpu as pltpu


def _body(m_ref, x_ref, w_ref, adj_ref, b_ref, out_ref, v_ref, at_ref):
    T = at_ref.shape[0]
    n = pl.program_id(0)
    t = pl.program_id(1)

    @pl.when(jnp.logical_and(n == 0, t == 0))
    def _prep():
        for i in range(T):
            xt = m_ref[i, 0] * x_ref[0]
            for j in range(1, T):
                xt = xt + m_ref[i, j] * x_ref[j]
            v_ref[i] = jnp.dot(xt, w_ref[i], preferred_element_type=jnp.float32)

    adjb = adj_ref[0]

    @pl.when(t == 0)
    def _init():
        for i in range(T):
            at_ref[i] = m_ref[i, 0] * adjb

    @pl.when(jnp.logical_and(t > 0, t < T - 1))
    def _accum():
        for i in range(T):
            at_ref[i] = at_ref[i] + m_ref[i, t] * adjb

    @pl.when(t == T - 1)
    def _finish():
        for i in range(T):
            at = at_ref[i] + m_ref[i, T - 1] * adjb
            out_ref[i] = b_ref[i] + jnp.dot(
                at, v_ref[i], preferred_element_type=jnp.float32
            )


@jax.jit
def kernel(x, adj, M, W, b):
    T, N, D_IN = x.shape
    D_OUT = W.shape[2]
    BN = min(256, N)

    out = pl.pallas_call(
        _body,
        grid=(N // BN, T),
        out_shape=jax.ShapeDtypeStruct((T, N, D_OUT), jnp.float32),
        in_specs=[
            pl.BlockSpec(memory_space=pltpu.SMEM),
            pl.BlockSpec((T, N, D_IN), lambda n, t: (0, 0, 0)),
            pl.BlockSpec((T, D_IN, D_OUT), lambda n, t: (0, 0, 0)),
            pl.BlockSpec((1, BN, N), lambda n, t: (t, n, 0)),
            pl.BlockSpec((T, BN, D_OUT), lambda n, t: (0, n, 0)),
        ],
        out_specs=pl.BlockSpec((T, BN, D_OUT), lambda n, t: (0, n, 0)),
        scratch_shapes=[
            pltpu.VMEM((T, N, D_OUT), jnp.float32),
            pltpu.VMEM((T, BN, N), jnp.float32),
        ],
        compiler_params=pltpu.CompilerParams(
            dimension_semantics=("arbitrary", "arbitrary"),
        ),
    )(M, x, W, adj, b)
    return out


# speedup vs baseline: 1.6299x; 1.6299x over previous
import jax
import jax.numpy as jnp
from jax.experimental import pallas as pl
from jax.experimental.pallas import tpu as pltpu


def _body(m_ref, x_ref, w_ref, a0_ref, a1_ref, a2_ref, a3_ref, b_ref, out_ref,
          v_ref):
    T = b_ref.shape[0]
    n = pl.program_id(0)

    @pl.when(n == 0)
    def _prep():
        for i in range(T):
            xt = m_ref[i, 0] * x_ref[0]
            for j in range(1, T):
                xt = xt + m_ref[i, j] * x_ref[j]
            v_ref[i] = jnp.dot(xt, w_ref[i], preferred_element_type=jnp.float32)

    slabs = [a0_ref[0], a1_ref[0], a2_ref[0], a3_ref[0]]
    for i in range(T):
        at = m_ref[i, 0] * slabs[0]
        for j in range(1, T):
            at = at + m_ref[i, j] * slabs[j]
        out_ref[i] = b_ref[i] + jnp.dot(
            at, v_ref[i], preferred_element_type=jnp.float32
        )


@jax.jit
def kernel(x, adj, M, W, b):
    T, N, D_IN = x.shape
    D_OUT = W.shape[2]
    BN = min(256, N)

    adj_specs = [
        pl.BlockSpec((1, BN, N), lambda n, j=j: (j, n, 0)) for j in range(T)
    ]
    out = pl.pallas_call(
        _body,
        grid=(N // BN,),
        out_shape=jax.ShapeDtypeStruct((T, N, D_OUT), jnp.float32),
        in_specs=[
            pl.BlockSpec(memory_space=pltpu.SMEM),
            pl.BlockSpec((T, N, D_IN), lambda n: (0, 0, 0)),
            pl.BlockSpec((T, D_IN, D_OUT), lambda n: (0, 0, 0)),
            *adj_specs,
            pl.BlockSpec((T, BN, D_OUT), lambda n: (0, n, 0)),
        ],
        out_specs=pl.BlockSpec((T, BN, D_OUT), lambda n: (0, n, 0)),
        scratch_shapes=[pltpu.VMEM((T, N, D_OUT), jnp.float32)],
        compiler_params=pltpu.CompilerParams(
            dimension_semantics=("arbitrary",),
        ),
    )(M, x, W, adj, adj, adj, adj, b)
    return out
